# baseline (speedup 1.0000x reference)
import functools

import jax
import jax.numpy as jnp
from jax import lax
from jax.experimental import pallas as pl
from jax.experimental.pallas import tpu as pltpu
from jax.experimental.pallas import tpu_sc as plsc

_ROWS = 1002
_EMBED = 128
_BATCH = 16384
_HIST = 200

_B = _BATCH * _HIST
_GC = 128
_SG = 2
_NB = 2
_KJ = 16
_NCHUNK = _B // _GC


def _make_gather():
    info = plsc.get_sparse_core_info()
    nc, ns = info.num_cores, info.num_subcores
    nw = nc * ns
    chunks_per_w = _NCHUNK // nw
    cps = _NB * _SG
    nsteps = chunks_per_w // cps
    steps_per_block = _KJ // cps
    mesh = plsc.VectorSubcoreMesh(core_axis_name="c", subcore_axis_name="s")

    @functools.partial(
        pl.kernel,
        mesh=mesh,
        out_type=jax.ShapeDtypeStruct((_B, _EMBED), jnp.float32),
        scratch_types=[
            pltpu.VMEM((_KJ, _GC), jnp.int32),
            pltpu.VMEM((_NB, _SG * _GC, _EMBED), jnp.float32),
            pltpu.VMEM_SHARED((_ROWS, _EMBED), jnp.float32),
        ]
        + [pltpu.SemaphoreType.DMA] * (2 * _NB),
    )
    def gather(idx_hbm, table_hbm, out_hbm, idx_v, rows_v, tab_sp, *sems):
        gsems, ssems = sems[:_NB], sems[_NB:]
        sid = lax.axis_index("s")
        wid = sid * nc + lax.axis_index("c")
        chunk0 = wid * chunks_per_w

        @pl.when(sid == 0)
        def _():
            pltpu.sync_copy(table_hbm, tab_sp)

        plsc.subcore_barrier()

        def step(t, carry):
            @pl.when(t % steps_per_block == 0)
            def _():
                blk = chunk0 + (t // steps_per_block) * _KJ
                pltpu.sync_copy(idx_hbm.at[pl.ds(blk, _KJ)], idx_v)

            for p in range(_NB):
                c = chunk0 + cps * t + _SG * p
                dst = out_hbm.at[pl.ds(c * _GC, _SG * _GC)]

                @pl.when(t > 0)
                def _():
                    pltpu.make_async_copy(rows_v.at[p], dst, ssems[p]).wait()

                for q in range(_SG):
                    row = idx_v.at[(cps * t) % _KJ + _SG * p + q]
                    seg = rows_v.at[p, pl.ds(q * _GC, _GC)]
                    pltpu.make_async_copy(tab_sp.at[row], seg, gsems[p]).start()

            for p in range(_NB):
                c = chunk0 + cps * t + _SG * p
                for q in range(_SG):
                    row = idx_v.at[(cps * t) % _KJ + _SG * p + q]
                    seg = rows_v.at[p, pl.ds(q * _GC, _GC)]
                    pltpu.make_async_copy(tab_sp.at[row], seg, gsems[p]).wait()
                dst = out_hbm.at[pl.ds(c * _GC, _SG * _GC)]
                pltpu.make_async_copy(rows_v.at[p], dst, ssems[p]).start()
            return carry

        lax.fori_loop(0, nsteps, step, 0)
        for p in range(_NB):
            c = chunk0 + cps * (nsteps - 1) + _SG * p
            dst = out_hbm.at[pl.ds(c * _GC, _SG * _GC)]
            pltpu.make_async_copy(rows_v.at[p], dst, ssems[p]).wait()

    return gather


def kernel(x, table):
    idx = x.astype(jnp.int32).reshape(_NCHUNK, _GC)
    out = _make_gather()(idx, table)
    return out.reshape(_BATCH, _HIST, _EMBED)


# speedup vs baseline: 1.4677x; 1.4677x over previous
import functools

import jax
import jax.numpy as jnp
from jax import lax
from jax.experimental import pallas as pl
from jax.experimental.pallas import tpu as pltpu
from jax.experimental.pallas import tpu_sc as plsc

_ROWS = 1002
_EMBED = 128
_BATCH = 16384
_HIST = 200

_B = _BATCH * _HIST
_GC = 80
_KJ = 32
_NB = 8
_NCHUNK = _B // _GC


def _make_gather():
    info = plsc.get_sparse_core_info()
    nc, ns = info.num_cores, info.num_subcores
    nw = nc * ns
    chunks_per_w = _NCHUNK // nw
    nsteps = chunks_per_w // _NB
    steps_per_block = _KJ // _NB
    mesh = plsc.VectorSubcoreMesh(core_axis_name="c", subcore_axis_name="s")

    @functools.partial(
        pl.kernel,
        mesh=mesh,
        out_type=jax.ShapeDtypeStruct((_B, _EMBED), jnp.float32),
        scratch_types=[
            pltpu.VMEM((_KJ, _GC), jnp.int32),
            pltpu.VMEM((_NB, _GC, _EMBED), jnp.float32),
            pltpu.VMEM_SHARED((_ROWS, _EMBED), jnp.float32),
        ]
        + [pltpu.SemaphoreType.DMA] * (2 * _NB),
    )
    def gather(idx_hbm, table_hbm, out_hbm, idx_v, rows_v, tab_sp, *sems):
        gsems, ssems = sems[:_NB], sems[_NB:]
        sid = lax.axis_index("s")
        wid = sid * nc + lax.axis_index("c")
        chunk0 = wid * chunks_per_w

        @pl.when(sid == 0)
        def _():
            pltpu.sync_copy(table_hbm, tab_sp)

        plsc.subcore_barrier()

        def step(t, carry):
            @pl.when(t % steps_per_block == 0)
            def _():
                blk = chunk0 + (t // steps_per_block) * _KJ
                pltpu.sync_copy(idx_hbm.at[pl.ds(blk, _KJ)], idx_v)

            for p in range(_NB):
                c = chunk0 + _NB * t + p
                rows_p = rows_v.at[p]
                dst = out_hbm.at[pl.ds(c * _GC, _GC)]

                @pl.when(t > 0)
                def _():
                    pltpu.make_async_copy(rows_p, dst, ssems[p]).wait()

                row = idx_v.at[(_NB * t) % _KJ + p]
                pltpu.make_async_copy(tab_sp.at[row], rows_p, gsems[p]).start()

            for p in range(_NB):
                c = chunk0 + _NB * t + p
                rows_p = rows_v.at[p]
                row = idx_v.at[(_NB * t) % _KJ + p]
                pltpu.make_async_copy(tab_sp.at[row], rows_p, gsems[p]).wait()
                dst = out_hbm.at[pl.ds(c * _GC, _GC)]
                pltpu.make_async_copy(rows_p, dst, ssems[p]).start()
            return carry

        lax.fori_loop(0, nsteps, step, 0)
        for p in range(_NB):
            c = chunk0 + _NB * (nsteps - 1) + p
            dst = out_hbm.at[pl.ds(c * _GC, _GC)]
            pltpu.make_async_copy(rows_v.at[p], dst, ssems[p]).wait()

    return gather


def kernel(x, table):
    idx = x.astype(jnp.int32).reshape(_NCHUNK, _GC)
    out = _make_gather()(idx, table)
    return out.reshape(_BATCH, _HIST, _EMBED)


# speedup vs baseline: 1.4772x; 1.0065x over previous
import functools

import jax
import jax.numpy as jnp
from jax import lax
from jax.experimental import pallas as pl
from jax.experimental.pallas import tpu as pltpu
from jax.experimental.pallas import tpu_sc as plsc

_ROWS = 1002
_EMBED = 128
_BATCH = 16384
_HIST = 200

_B = _BATCH * _HIST
_GC = 128
_KJ = 160
_NB = 5
_NCHUNK = _B // _GC


def _make_gather():
    info = plsc.get_sparse_core_info()
    nc, ns = info.num_cores, info.num_subcores
    nw = nc * ns
    chunks_per_w = _NCHUNK // nw
    nsteps = chunks_per_w // _NB
    steps_per_block = _KJ // _NB
    mesh = plsc.VectorSubcoreMesh(core_axis_name="c", subcore_axis_name="s")

    @functools.partial(
        pl.kernel,
        mesh=mesh,
        out_type=jax.ShapeDtypeStruct((_B, _EMBED), jnp.float32),
        scratch_types=[
            pltpu.VMEM((_KJ, _GC), jnp.int32),
            pltpu.VMEM((_NB, _GC, _EMBED), jnp.float32),
            pltpu.VMEM_SHARED((_ROWS, _EMBED), jnp.float32),
        ]
        + [pltpu.SemaphoreType.DMA] * (2 * _NB),
    )
    def gather(idx_hbm, table_hbm, out_hbm, idx_v, rows_v, tab_sp, *sems):
        gsems, ssems = sems[:_NB], sems[_NB:]
        sid = lax.axis_index("s")
        wid = sid * nc + lax.axis_index("c")
        chunk0 = wid * chunks_per_w

        @pl.when(sid == 0)
        def _():
            pltpu.sync_copy(table_hbm, tab_sp)

        plsc.subcore_barrier()

        def step(t, carry):
            @pl.when(t % steps_per_block == 0)
            def _():
                blk = chunk0 + (t // steps_per_block) * _KJ
                pltpu.sync_copy(idx_hbm.at[pl.ds(blk, _KJ)], idx_v)

            for p in range(_NB):
                c = chunk0 + _NB * t + p
                rows_p = rows_v.at[p]
                dst = out_hbm.at[pl.ds(c * _GC, _GC)]

                @pl.when(t > 0)
                def _():
                    pltpu.make_async_copy(rows_p, dst, ssems[p]).wait()

                row = idx_v.at[(_NB * t) % _KJ + p]
                pltpu.make_async_copy(tab_sp.at[row], rows_p, gsems[p]).start()

            for p in range(_NB):
                c = chunk0 + _NB * t + p
                rows_p = rows_v.at[p]
                row = idx_v.at[(_NB * t) % _KJ + p]
                pltpu.make_async_copy(tab_sp.at[row], rows_p, gsems[p]).wait()
                dst = out_hbm.at[pl.ds(c * _GC, _GC)]
                pltpu.make_async_copy(rows_p, dst, ssems[p]).start()
            return carry

        lax.fori_loop(0, nsteps, step, 0)
        for p in range(_NB):
            c = chunk0 + _NB * (nsteps - 1) + p
            dst = out_hbm.at[pl.ds(c * _GC, _GC)]
            pltpu.make_async_copy(rows_v.at[p], dst, ssems[p]).wait()

    return gather


def kernel(x, table):
    idx = x.astype(jnp.int32).reshape(_NCHUNK, _GC)
    out = _make_gather()(idx, table)
    return out.reshape(_BATCH, _HIST, _EMBED)


# speedup vs baseline: 1.4833x; 1.0041x over previous
import functools

import jax
import jax.numpy as jnp
from jax import lax
from jax.experimental import pallas as pl
from jax.experimental.pallas import tpu as pltpu
from jax.experimental.pallas import tpu_sc as plsc

_ROWS = 1002
_EMBED = 128
_BATCH = 16384
_HIST = 200

_B = _BATCH * _HIST
_GC = 128
_KJ = 80
_NB = 5
_NCHUNK = _B // _GC


def _make_gather():
    info = plsc.get_sparse_core_info()
    nc, ns = info.num_cores, info.num_subcores
    nw = nc * ns
    chunks_per_w = _NCHUNK // nw
    nsteps = chunks_per_w // _NB
    steps_per_block = _KJ // _NB
    nblocks = chunks_per_w // _KJ
    mesh = plsc.VectorSubcoreMesh(core_axis_name="c", subcore_axis_name="s")

    @functools.partial(
        pl.kernel,
        mesh=mesh,
        out_type=jax.ShapeDtypeStruct((_B, _EMBED), jnp.float32),
        scratch_types=[
            pltpu.VMEM((2 * _KJ, _GC), jnp.int32),
            pltpu.VMEM((_NB, _GC, _EMBED), jnp.float32),
            pltpu.VMEM_SHARED((_ROWS, _EMBED), jnp.float32),
        ]
        + [pltpu.SemaphoreType.DMA] * (1 + 2 * _NB),
    )
    def gather(idx_hbm, table_hbm, out_hbm, idx_v, rows_v, tab_sp, *sems):
        isem, gsems, ssems = sems[0], sems[1 : 1 + _NB], sems[1 + _NB :]
        sid = lax.axis_index("s")
        wid = sid * nc + lax.axis_index("c")
        chunk0 = wid * chunks_per_w

        @pl.when(sid == 0)
        def _():
            pltpu.sync_copy(table_hbm, tab_sp)

        pltpu.make_async_copy(
            idx_hbm.at[pl.ds(chunk0, _KJ)], idx_v.at[pl.ds(0, _KJ)], isem
        ).start()
        plsc.subcore_barrier()

        def step(t, carry):
            bi = t // steps_per_block

            @pl.when(t % steps_per_block == 0)
            def _():
                off_cur = pl.multiple_of(lax.rem(bi, 2) * _KJ, 8)
                off_nxt = pl.multiple_of(lax.rem(bi + 1, 2) * _KJ, 8)
                pltpu.make_async_copy(
                    idx_hbm.at[pl.ds(chunk0 + bi * _KJ, _KJ)],
                    idx_v.at[pl.ds(off_cur, _KJ)],
                    isem,
                ).wait()

                @pl.when(bi + 1 < nblocks)
                def _():
                    pltpu.make_async_copy(
                        idx_hbm.at[pl.ds(chunk0 + (bi + 1) * _KJ, _KJ)],
                        idx_v.at[pl.ds(off_nxt, _KJ)],
                        isem,
                    ).start()

            for p in range(_NB):
                c = chunk0 + _NB * t + p
                rows_p = rows_v.at[p]
                dst = out_hbm.at[pl.ds(c * _GC, _GC)]

                @pl.when(t > 0)
                def _():
                    pltpu.make_async_copy(rows_p, dst, ssems[p]).wait()

                row = idx_v.at[(_NB * t) % (2 * _KJ) + p]
                pltpu.make_async_copy(tab_sp.at[row], rows_p, gsems[p]).start()

            for p in range(_NB):
                c = chunk0 + _NB * t + p
                rows_p = rows_v.at[p]
                row = idx_v.at[(_NB * t) % (2 * _KJ) + p]
                pltpu.make_async_copy(tab_sp.at[row], rows_p, gsems[p]).wait()
                dst = out_hbm.at[pl.ds(c * _GC, _GC)]
                pltpu.make_async_copy(rows_p, dst, ssems[p]).start()
            return carry

        lax.fori_loop(0, nsteps, step, 0)
        for p in range(_NB):
            c = chunk0 + _NB * (nsteps - 1) + p
            dst = out_hbm.at[pl.ds(c * _GC, _GC)]
            pltpu.make_async_copy(rows_v.at[p], dst, ssems[p]).wait()

    return gather


def kernel(x, table):
    idx = x.astype(jnp.int32).reshape(_NCHUNK, _GC)
    out = _make_gather()(idx, table)
    return out.reshape(_BATCH, _HIST, _EMBED)
